# baseline (speedup 1.0000x reference)
import functools

import jax
import jax.numpy as jnp
from jax import lax
from jax.experimental import pallas as pl
from jax.experimental.pallas import tpu as pltpu
from jax.experimental.pallas import tpu_sc as plsc


def kernel(x, embed_table):
    b, n, m = x.shape
    e = embed_table.shape[-1]
    info = plsc.get_sparse_core_info()
    nw = info.num_cores * info.num_subcores
    per_w = b // nw

    mesh = plsc.VectorSubcoreMesh(core_axis_name="c", subcore_axis_name="s")

    @functools.partial(
        pl.kernel,
        out_type=jax.ShapeDtypeStruct((b, n, m + e), jnp.float32),
        mesh=mesh,
        scratch_types=[pltpu.VMEM((n, m + e), jnp.float32)],
        compiler_params=pltpu.CompilerParams(use_tc_tiling_on_sc=False),
    )
    def run(x_hbm, tab_hbm, out_hbm, buf):
        wid = lax.axis_index("s") * info.num_cores + lax.axis_index("c")
        base = wid * per_w
        pltpu.sync_copy(tab_hbm, buf.at[:, pl.ds(m, e)])

        def body(i, carry):
            bi = base + i
            pltpu.sync_copy(x_hbm.at[bi], buf.at[:, pl.ds(0, m)])
            pltpu.sync_copy(buf, out_hbm.at[bi])
            return carry

        lax.fori_loop(0, per_w, body, 0)

    return run(x, embed_table)
